# baseline (speedup 1.0000x reference)
import functools

import jax
import jax.numpy as jnp
from jax import lax
from jax.experimental import pallas as pl
from jax.experimental.pallas import tpu as pltpu
from jax.experimental.pallas import tpu_sc as plsc

N = 10000
E = 320000
D_IN = 128
D_H = 128
D_OUT = 64

NC = 2
NS = 16
NW = NC * NS

B = 128
CPW0 = 40
CPW1 = 120
HCH = 40
EPAD = NS * (CPW0 + CPW1) * B
NPAD = 10240
RPT = N // NS
ZPT = NPAD // NS


def _seg_sum_body(with_deg, D, *refs):
    if with_deg:
        (src_hbm, dst_hbm, table_hbm, out_hbm, deg_hbm,
         src_v, dst_v, buf0, buf1, sem0, sem1, acc,
         ones_v, dz_v, dacc) = refs
    else:
        (src_hbm, dst_hbm, table_hbm, out_hbm,
         src_v, dst_v, buf0, buf1, sem0, sem1, acc) = refs

    c = lax.axis_index("c")
    s = lax.axis_index("s")
    _Z16 = jnp.zeros((16,), jnp.float32)
    _O16 = jnp.ones((16,), jnp.float32)

    def _zrow(i, carry):
        for k in range(D // 16):
            buf0[i, pl.ds(k * 16, 16)] = _Z16
        return carry
    lax.fori_loop(0, B, _zrow, 0)
    for t in range(ZPT // B):
        pltpu.sync_copy(buf0, acc.at[pl.ds(s * ZPT + t * B, B)])

    if with_deg:
        def _z1(i, carry):
            dz_v[pl.ds(i * 16, 16)] = _Z16
            return carry
        lax.fori_loop(0, ZPT // 16, _z1, 0)
        pltpu.sync_copy(dz_v, dacc.at[pl.ds(s * ZPT, ZPT)])
        for k in range(B // 16):
            ones_v[pl.ds(k * 16, 16)] = _O16

    plsc.subcore_barrier()

    def _half(g, j, buf, sem):
        pltpu.make_async_copy(table_hbm.at[src_v.at[0]], buf, sem).wait()
        pltpu.sync_copy(buf, acc.at[dst_v.at[j]], add=True)
        if with_deg:
            pltpu.sync_copy(ones_v, dacc.at[dst_v.at[j]], add=True)

        @pl.when(g < HCH // 2 - 1)
        def _():
            pltpu.async_copy(table_hbm.at[src_v.at[j + 2]], buf, sem)

    def _step(g, carry):
        _half(g, 2 * g, buf0, sem0)
        _half(g, 2 * g + 1, buf1, sem1)
        return carry

    base_c = jnp.where(c == 0, s * CPW0, NS * CPW0 + s * CPW1)
    nph_c = jnp.where(c == 0, CPW0 // HCH, CPW1 // HCH)
    for phase in range(max(CPW0, CPW1) // HCH):
        @pl.when(phase < nph_c)
        def _():
            pltpu.sync_copy(src_hbm.at[pl.ds(base_c + phase * HCH, HCH)], src_v)
            pltpu.sync_copy(dst_hbm.at[pl.ds(base_c + phase * HCH, HCH)], dst_v)
            pltpu.async_copy(table_hbm.at[src_v.at[0]], buf0, sem0)
            pltpu.async_copy(table_hbm.at[src_v.at[1]], buf1, sem1)
            lax.fori_loop(0, HCH // 2, _step, 0)

    plsc.subcore_barrier()

    pltpu.sync_copy(acc.at[pl.ds(s * ZPT, ZPT)],
                    out_hbm.at[pl.ds(c * NPAD + s * ZPT, ZPT)])
    if with_deg:
        pltpu.sync_copy(dacc.at[pl.ds(s * ZPT, ZPT)],
                        deg_hbm.at[pl.ds(c * NPAD + s * ZPT, ZPT)])


def _make_seg_sum(D, with_deg):
    out_type = [jax.ShapeDtypeStruct((NC * NPAD, D), jnp.float32)]
    scratch = [
        pltpu.VMEM((HCH, B), jnp.int32),
        pltpu.VMEM((HCH, B), jnp.int32),
        pltpu.VMEM((B, D), jnp.float32),
        pltpu.VMEM((B, D), jnp.float32),
        pltpu.SemaphoreType.DMA,
        pltpu.SemaphoreType.DMA,
        pltpu.VMEM_SHARED((NPAD, D), jnp.float32),
    ]
    if with_deg:
        out_type.append(jax.ShapeDtypeStruct((NC * NPAD,), jnp.float32))
        scratch += [
            pltpu.VMEM((B,), jnp.float32),
            pltpu.VMEM((ZPT,), jnp.float32),
            pltpu.VMEM_SHARED((NPAD,), jnp.float32),
        ]
    mesh = plsc.VectorSubcoreMesh(core_axis_name="c", subcore_axis_name="s")
    return pl.kernel(
        functools.partial(_seg_sum_body, with_deg, D),
        out_type=out_type,
        mesh=mesh,
        scratch_types=scratch,
    )


@functools.lru_cache(maxsize=None)
def _get_seg_sum(D, with_deg):
    return _make_seg_sum(D, with_deg)


RB = 400
GRID = N // RB


def _rows(i):
    return (i, 0)


def _full(i):
    return (0, 0)


def _pair(i):
    return (0, i, 0)


def _tc_call(body, out_shapes, in_specs, out_specs, args):
    return pl.pallas_call(
        body,
        grid=(GRID,),
        in_specs=in_specs,
        out_specs=out_specs,
        out_shape=out_shapes,
    )(*args)


def _k_pre(x, w_cat):
    def body(x_ref, w_ref, a_ref, b_ref):
        res = jnp.dot(x_ref[...], w_ref[...], preferred_element_type=jnp.float32)
        a_ref[...] = res[:, :D_H]
        b_ref[...] = res[:, D_H:]
    return _tc_call(
        body,
        [jax.ShapeDtypeStruct((N, D_H), jnp.float32)] * 2,
        [pl.BlockSpec((RB, D_IN), _rows), pl.BlockSpec((D_IN, 2 * D_H), _full)],
        [pl.BlockSpec((RB, D_H), _rows)] * 2,
        (x, w_cat),
    )


def _k_sage1_gcnpre(p1, dp, r1, b1, wg, bg):
    def body(p_ref, d_ref, r_ref, b_ref, wg_ref, bg_ref,
             xwp_ref, self_ref, deg_ref):
        deg = d_ref[0, :, 0] + d_ref[1, :, 0]
        degc = jnp.maximum(deg, 1.0)
        mean = (p_ref[0] + p_ref[1]) / degc[:, None]
        h1 = jnp.maximum(mean + r_ref[...] + b_ref[...], 0.0)
        xw = jnp.dot(h1, wg_ref[...], preferred_element_type=jnp.float32)
        invs = lax.rsqrt(deg + 1.0)
        xwp_ref[...] = xw * invs[:, None]
        self_ref[...] = xw * (invs * invs)[:, None] + bg_ref[...]
        deg_ref[...] = deg[:, None]
    return _tc_call(
        body,
        [jax.ShapeDtypeStruct((N, D_H), jnp.float32),
         jax.ShapeDtypeStruct((N, D_H), jnp.float32),
         jax.ShapeDtypeStruct((N, 1), jnp.float32)],
        [pl.BlockSpec((2, RB, D_H), _pair),
         pl.BlockSpec((2, RB, 1), _pair),
         pl.BlockSpec((RB, D_H), _rows),
         pl.BlockSpec((1, D_H), _full),
         pl.BlockSpec((D_H, D_H), _full),
         pl.BlockSpec((1, D_H), _full)],
        [pl.BlockSpec((RB, D_H), _rows),
         pl.BlockSpec((RB, D_H), _rows),
         pl.BlockSpec((RB, 1), _rows)],
        (p1, dp, r1, b1, wg, bg),
    )


def _k_gcn_sage3pre(p2, deg, self2, w_cat):
    def body(p_ref, d_ref, s_ref, w_ref, a_ref, b_ref):
        deg = d_ref[..., 0]
        invs = lax.rsqrt(deg + 1.0)
        h2 = jnp.maximum(invs[:, None] * (p_ref[0] + p_ref[1]) + s_ref[...], 0.0)
        res = jnp.dot(h2, w_ref[...], preferred_element_type=jnp.float32)
        a_ref[...] = res[:, :D_H]
        b_ref[...] = res[:, D_H:]
    return _tc_call(
        body,
        [jax.ShapeDtypeStruct((N, D_H), jnp.float32)] * 2,
        [pl.BlockSpec((2, RB, D_H), _pair),
         pl.BlockSpec((RB, 1), _rows),
         pl.BlockSpec((RB, D_H), _rows),
         pl.BlockSpec((D_H, 2 * D_H), _full)],
        [pl.BlockSpec((RB, D_H), _rows)] * 2,
        (p2, deg, self2, w_cat),
    )


def _k_sage3_sage4pre(p3, deg, r3, b3, wr4):
    def body(p_ref, d_ref, r_ref, b_ref, w_ref, h_ref, c_ref):
        degc = jnp.maximum(d_ref[..., 0], 1.0)
        h3 = jnp.maximum((p_ref[0] + p_ref[1]) / degc[:, None]
                         + r_ref[...] + b_ref[...], 0.0)
        h_ref[...] = h3
        c_ref[...] = jnp.dot(h3, w_ref[...], preferred_element_type=jnp.float32)
    return _tc_call(
        body,
        [jax.ShapeDtypeStruct((N, D_H), jnp.float32),
         jax.ShapeDtypeStruct((N, D_OUT), jnp.float32)],
        [pl.BlockSpec((2, RB, D_H), _pair),
         pl.BlockSpec((RB, 1), _rows),
         pl.BlockSpec((RB, D_H), _rows),
         pl.BlockSpec((1, D_H), _full),
         pl.BlockSpec((D_H, D_OUT), _full)],
        [pl.BlockSpec((RB, D_H), _rows),
         pl.BlockSpec((RB, D_OUT), _rows)],
        (p3, deg, r3, b3, wr4),
    )


def _k_out(p4, deg, r4, b4, wl4):
    def body(p_ref, d_ref, r_ref, b_ref, w_ref, o_ref):
        degc = jnp.maximum(d_ref[..., 0], 1.0)
        mean = (p_ref[0] + p_ref[1]) / degc[:, None]
        z = (jnp.dot(mean, w_ref[...], preferred_element_type=jnp.float32)
             + r_ref[...] + b_ref[...])
        m = jnp.max(z, axis=1, keepdims=True)
        lse = jnp.log(jnp.sum(jnp.exp(z - m), axis=1, keepdims=True)) + m
        o_ref[...] = z - lse
    return _tc_call(
        body,
        jax.ShapeDtypeStruct((N, D_OUT), jnp.float32),
        [pl.BlockSpec((2, RB, D_H), _pair),
         pl.BlockSpec((RB, 1), _rows),
         pl.BlockSpec((RB, D_OUT), _rows),
         pl.BlockSpec((1, D_OUT), _full),
         pl.BlockSpec((D_H, D_OUT), _full)],
        pl.BlockSpec((RB, D_OUT), _rows),
        (p4, deg, r4, b4, wl4),
    )


def kernel(x, edge_index, Wl1, Wr1, b1, Wg, bg, Wl3, Wr3, b3, Wl4, Wr4, b4):
    src = edge_index[0].astype(jnp.int32)
    dst = edge_index[1].astype(jnp.int32)
    src_p = jnp.concatenate(
        [src, jnp.zeros((EPAD - E,), jnp.int32)]).reshape(EPAD // B, B)
    dst_p = jnp.concatenate(
        [dst, jnp.full((EPAD - E,), N, jnp.int32)]).reshape(EPAD // B, B)

    b1r = b1.reshape(1, D_H)
    bgr = bg.reshape(1, D_H)
    b3r = b3.reshape(1, D_H)
    b4r = b4.reshape(1, D_OUT)

    y1l, r1 = _k_pre(x, jnp.concatenate([Wl1, Wr1], axis=1))
    p1_flat, degp = _get_seg_sum(D_H, True)(src_p, dst_p, y1l)
    p1 = p1_flat.reshape(NC, NPAD, D_H)
    dp = degp.reshape(NC, NPAD)[:, :, None]

    xw_pre, self2, deg = _k_sage1_gcnpre(p1, dp, r1, b1r, Wg, bgr)

    (p2_flat,) = _get_seg_sum(D_H, False)(src_p, dst_p, xw_pre)
    p2 = p2_flat.reshape(NC, NPAD, D_H)

    y3l, r3 = _k_gcn_sage3pre(p2, deg, self2, jnp.concatenate([Wl3, Wr3], axis=1))

    (p3_flat,) = _get_seg_sum(D_H, False)(src_p, dst_p, y3l)
    p3 = p3_flat.reshape(NC, NPAD, D_H)

    h3, r4 = _k_sage3_sage4pre(p3, deg, r3, b3r, Wr4)

    (p4_flat,) = _get_seg_sum(D_H, False)(src_p, dst_p, h3)
    p4 = p4_flat.reshape(NC, NPAD, D_H)

    return _k_out(p4, deg, r4, b4r, Wl4)


# speedup vs baseline: 1.0764x; 1.0764x over previous
import functools

import jax
import jax.numpy as jnp
from jax import lax
from jax.experimental import pallas as pl
from jax.experimental.pallas import tpu as pltpu
from jax.experimental.pallas import tpu_sc as plsc

N = 10000
E = 320000
D_IN = 128
D_H = 128
D_OUT = 64

NC = 2
NS = 16
NW = NC * NS

B = 128
CPW0 = 120
CPW1 = 40
HCH = 40
EPAD = NS * (CPW0 + CPW1) * B
NPAD = 10240
RPT = N // NS
ZPT = NPAD // NS


def _seg_sum_body(with_deg, D, *refs):
    if with_deg:
        (src_hbm, dst_hbm, table_hbm, out_hbm, deg_hbm,
         src_v, dst_v, buf0, buf1, sem0, sem1, acc,
         ones_v, dz_v, dacc) = refs
    else:
        (src_hbm, dst_hbm, table_hbm, out_hbm,
         src_v, dst_v, buf0, buf1, sem0, sem1, acc) = refs

    c = lax.axis_index("c")
    s = lax.axis_index("s")
    _Z16 = jnp.zeros((16,), jnp.float32)
    _O16 = jnp.ones((16,), jnp.float32)

    def _zrow(i, carry):
        for k in range(D // 16):
            buf0[i, pl.ds(k * 16, 16)] = _Z16
        return carry
    lax.fori_loop(0, B, _zrow, 0)
    for t in range(ZPT // B):
        pltpu.sync_copy(buf0, acc.at[pl.ds(s * ZPT + t * B, B)])

    if with_deg:
        def _z1(i, carry):
            dz_v[pl.ds(i * 16, 16)] = _Z16
            return carry
        lax.fori_loop(0, ZPT // 16, _z1, 0)
        pltpu.sync_copy(dz_v, dacc.at[pl.ds(s * ZPT, ZPT)])
        for k in range(B // 16):
            ones_v[pl.ds(k * 16, 16)] = _O16

    plsc.subcore_barrier()

    def _half(g, j, buf, sem):
        pltpu.make_async_copy(table_hbm.at[src_v.at[0]], buf, sem).wait()
        pltpu.sync_copy(buf, acc.at[dst_v.at[j]], add=True)
        if with_deg:
            pltpu.sync_copy(ones_v, dacc.at[dst_v.at[j]], add=True)

        @pl.when(g < HCH // 2 - 1)
        def _():
            pltpu.async_copy(table_hbm.at[src_v.at[j + 2]], buf, sem)

    def _step(g, carry):
        _half(g, 2 * g, buf0, sem0)
        _half(g, 2 * g + 1, buf1, sem1)
        return carry

    base_c = jnp.where(c == 0, s * CPW0, NS * CPW0 + s * CPW1)
    nph_c = jnp.where(c == 0, CPW0 // HCH, CPW1 // HCH)
    for phase in range(max(CPW0, CPW1) // HCH):
        @pl.when(phase < nph_c)
        def _():
            pltpu.sync_copy(src_hbm.at[pl.ds(base_c + phase * HCH, HCH)], src_v)
            pltpu.sync_copy(dst_hbm.at[pl.ds(base_c + phase * HCH, HCH)], dst_v)
            pltpu.async_copy(table_hbm.at[src_v.at[0]], buf0, sem0)
            pltpu.async_copy(table_hbm.at[src_v.at[1]], buf1, sem1)
            lax.fori_loop(0, HCH // 2, _step, 0)

    plsc.subcore_barrier()

    pltpu.sync_copy(acc.at[pl.ds(s * ZPT, ZPT)],
                    out_hbm.at[pl.ds(c * NPAD + s * ZPT, ZPT)])
    if with_deg:
        pltpu.sync_copy(dacc.at[pl.ds(s * ZPT, ZPT)],
                        deg_hbm.at[pl.ds(c * NPAD + s * ZPT, ZPT)])


def _make_seg_sum(D, with_deg):
    out_type = [jax.ShapeDtypeStruct((NC * NPAD, D), jnp.float32)]
    scratch = [
        pltpu.VMEM((HCH, B), jnp.int32),
        pltpu.VMEM((HCH, B), jnp.int32),
        pltpu.VMEM((B, D), jnp.float32),
        pltpu.VMEM((B, D), jnp.float32),
        pltpu.SemaphoreType.DMA,
        pltpu.SemaphoreType.DMA,
        pltpu.VMEM_SHARED((NPAD, D), jnp.float32),
    ]
    if with_deg:
        out_type.append(jax.ShapeDtypeStruct((NC * NPAD,), jnp.float32))
        scratch += [
            pltpu.VMEM((B,), jnp.float32),
            pltpu.VMEM((ZPT,), jnp.float32),
            pltpu.VMEM_SHARED((NPAD,), jnp.float32),
        ]
    mesh = plsc.VectorSubcoreMesh(core_axis_name="c", subcore_axis_name="s")
    return pl.kernel(
        functools.partial(_seg_sum_body, with_deg, D),
        out_type=out_type,
        mesh=mesh,
        scratch_types=scratch,
    )


@functools.lru_cache(maxsize=None)
def _get_seg_sum(D, with_deg):
    return _make_seg_sum(D, with_deg)


RB = 400
GRID = N // RB


def _rows(i):
    return (i, 0)


def _full(i):
    return (0, 0)


def _pair(i):
    return (0, i, 0)


def _tc_call(body, out_shapes, in_specs, out_specs, args):
    return pl.pallas_call(
        body,
        grid=(GRID,),
        in_specs=in_specs,
        out_specs=out_specs,
        out_shape=out_shapes,
    )(*args)


def _k_pre(x, w_cat):
    def body(x_ref, w_ref, a_ref, b_ref):
        res = jnp.dot(x_ref[...], w_ref[...], preferred_element_type=jnp.float32)
        a_ref[...] = res[:, :D_H]
        b_ref[...] = res[:, D_H:]
    return _tc_call(
        body,
        [jax.ShapeDtypeStruct((N, D_H), jnp.float32)] * 2,
        [pl.BlockSpec((RB, D_IN), _rows), pl.BlockSpec((D_IN, 2 * D_H), _full)],
        [pl.BlockSpec((RB, D_H), _rows)] * 2,
        (x, w_cat),
    )


def _k_sage1_gcnpre(p1, dp, r1, b1, wg, bg):
    def body(p_ref, d_ref, r_ref, b_ref, wg_ref, bg_ref,
             xwp_ref, self_ref, deg_ref):
        deg = d_ref[0, :, 0] + d_ref[1, :, 0]
        degc = jnp.maximum(deg, 1.0)
        mean = (p_ref[0] + p_ref[1]) / degc[:, None]
        h1 = jnp.maximum(mean + r_ref[...] + b_ref[...], 0.0)
        xw = jnp.dot(h1, wg_ref[...], preferred_element_type=jnp.float32)
        invs = lax.rsqrt(deg + 1.0)
        xwp_ref[...] = xw * invs[:, None]
        self_ref[...] = xw * (invs * invs)[:, None] + bg_ref[...]
        deg_ref[...] = deg[:, None]
    return _tc_call(
        body,
        [jax.ShapeDtypeStruct((N, D_H), jnp.float32),
         jax.ShapeDtypeStruct((N, D_H), jnp.float32),
         jax.ShapeDtypeStruct((N, 1), jnp.float32)],
        [pl.BlockSpec((2, RB, D_H), _pair),
         pl.BlockSpec((2, RB, 1), _pair),
         pl.BlockSpec((RB, D_H), _rows),
         pl.BlockSpec((1, D_H), _full),
         pl.BlockSpec((D_H, D_H), _full),
         pl.BlockSpec((1, D_H), _full)],
        [pl.BlockSpec((RB, D_H), _rows),
         pl.BlockSpec((RB, D_H), _rows),
         pl.BlockSpec((RB, 1), _rows)],
        (p1, dp, r1, b1, wg, bg),
    )


def _k_gcn_sage3pre(p2, deg, self2, w_cat):
    def body(p_ref, d_ref, s_ref, w_ref, a_ref, b_ref):
        deg = d_ref[..., 0]
        invs = lax.rsqrt(deg + 1.0)
        h2 = jnp.maximum(invs[:, None] * (p_ref[0] + p_ref[1]) + s_ref[...], 0.0)
        res = jnp.dot(h2, w_ref[...], preferred_element_type=jnp.float32)
        a_ref[...] = res[:, :D_H]
        b_ref[...] = res[:, D_H:]
    return _tc_call(
        body,
        [jax.ShapeDtypeStruct((N, D_H), jnp.float32)] * 2,
        [pl.BlockSpec((2, RB, D_H), _pair),
         pl.BlockSpec((RB, 1), _rows),
         pl.BlockSpec((RB, D_H), _rows),
         pl.BlockSpec((D_H, 2 * D_H), _full)],
        [pl.BlockSpec((RB, D_H), _rows)] * 2,
        (p2, deg, self2, w_cat),
    )


def _k_sage3_sage4pre(p3, deg, r3, b3, wr4):
    def body(p_ref, d_ref, r_ref, b_ref, w_ref, h_ref, c_ref):
        degc = jnp.maximum(d_ref[..., 0], 1.0)
        h3 = jnp.maximum((p_ref[0] + p_ref[1]) / degc[:, None]
                         + r_ref[...] + b_ref[...], 0.0)
        h_ref[...] = h3
        c_ref[...] = jnp.dot(h3, w_ref[...], preferred_element_type=jnp.float32)
    return _tc_call(
        body,
        [jax.ShapeDtypeStruct((N, D_H), jnp.float32),
         jax.ShapeDtypeStruct((N, D_OUT), jnp.float32)],
        [pl.BlockSpec((2, RB, D_H), _pair),
         pl.BlockSpec((RB, 1), _rows),
         pl.BlockSpec((RB, D_H), _rows),
         pl.BlockSpec((1, D_H), _full),
         pl.BlockSpec((D_H, D_OUT), _full)],
        [pl.BlockSpec((RB, D_H), _rows),
         pl.BlockSpec((RB, D_OUT), _rows)],
        (p3, deg, r3, b3, wr4),
    )


def _k_out(p4, deg, r4, b4, wl4):
    def body(p_ref, d_ref, r_ref, b_ref, w_ref, o_ref):
        degc = jnp.maximum(d_ref[..., 0], 1.0)
        mean = (p_ref[0] + p_ref[1]) / degc[:, None]
        z = (jnp.dot(mean, w_ref[...], preferred_element_type=jnp.float32)
             + r_ref[...] + b_ref[...])
        m = jnp.max(z, axis=1, keepdims=True)
        lse = jnp.log(jnp.sum(jnp.exp(z - m), axis=1, keepdims=True)) + m
        o_ref[...] = z - lse
    return _tc_call(
        body,
        jax.ShapeDtypeStruct((N, D_OUT), jnp.float32),
        [pl.BlockSpec((2, RB, D_H), _pair),
         pl.BlockSpec((RB, 1), _rows),
         pl.BlockSpec((RB, D_OUT), _rows),
         pl.BlockSpec((1, D_OUT), _full),
         pl.BlockSpec((D_H, D_OUT), _full)],
        pl.BlockSpec((RB, D_OUT), _rows),
        (p4, deg, r4, b4, wl4),
    )


def kernel(x, edge_index, Wl1, Wr1, b1, Wg, bg, Wl3, Wr3, b3, Wl4, Wr4, b4):
    src = edge_index[0].astype(jnp.int32)
    dst = edge_index[1].astype(jnp.int32)
    src_p = jnp.concatenate(
        [src, jnp.zeros((EPAD - E,), jnp.int32)]).reshape(EPAD // B, B)
    dst_p = jnp.concatenate(
        [dst, jnp.full((EPAD - E,), N, jnp.int32)]).reshape(EPAD // B, B)

    b1r = b1.reshape(1, D_H)
    bgr = bg.reshape(1, D_H)
    b3r = b3.reshape(1, D_H)
    b4r = b4.reshape(1, D_OUT)

    y1l, r1 = _k_pre(x, jnp.concatenate([Wl1, Wr1], axis=1))
    p1_flat, degp = _get_seg_sum(D_H, True)(src_p, dst_p, y1l)
    p1 = p1_flat.reshape(NC, NPAD, D_H)
    dp = degp.reshape(NC, NPAD)[:, :, None]

    xw_pre, self2, deg = _k_sage1_gcnpre(p1, dp, r1, b1r, Wg, bgr)

    (p2_flat,) = _get_seg_sum(D_H, False)(src_p, dst_p, xw_pre)
    p2 = p2_flat.reshape(NC, NPAD, D_H)

    y3l, r3 = _k_gcn_sage3pre(p2, deg, self2, jnp.concatenate([Wl3, Wr3], axis=1))

    (p3_flat,) = _get_seg_sum(D_H, False)(src_p, dst_p, y3l)
    p3 = p3_flat.reshape(NC, NPAD, D_H)

    h3, r4 = _k_sage3_sage4pre(p3, deg, r3, b3r, Wr4)

    (p4_flat,) = _get_seg_sum(D_H, False)(src_p, dst_p, h3)
    p4 = p4_flat.reshape(NC, NPAD, D_H)

    return _k_out(p4, deg, r4, b4r, Wl4)
